# baseline (speedup 1.0000x reference)
import functools

import jax
import jax.numpy as jnp
from jax import lax
from jax.experimental import pallas as pl
from jax.experimental.pallas import tpu as pltpu
from jax.experimental.pallas import tpu_sc as plsc

N = 10000
E = 320000
D = 128
K = 10

NC = 2
NS = 16
NW = NC * NS
EB = E // NW
EBP = EB + 240
NP = 10240
NPT = NP // NW
SLOTW = 16
NK = NP * SLOTW
SLOT_PAD = 8192
DUMROW = N


def _mesh():
    return plsc.VectorSubcoreMesh(
        core_axis_name="c", subcore_axis_name="s",
        num_cores=NC, num_subcores=NS)


def _wid():
    return lax.axis_index("s") * NC + lax.axis_index("c")


@functools.partial(
    pl.kernel,
    out_type=(jax.ShapeDtypeStruct((E,), jnp.int32),
              jax.ShapeDtypeStruct((NW * NP,), jnp.int32)),
    mesh=_mesh(),
    compiler_params=pltpu.CompilerParams(needs_layout_passes=False),
    scratch_types=[pltpu.VMEM((EB,), jnp.int32),
                   pltpu.VMEM((EB,), jnp.int32),
                   pltpu.VMEM((NP,), jnp.int32)])
def _k1(rows_hbm, lrank_hbm, blk_hbm, rows_v, lrank_v, cnt_v):
    wid = _wid()
    base = wid * EB
    pltpu.sync_copy(rows_hbm.at[pl.ds(base, EB)], rows_v)

    def zbody(i, _):
        cnt_v[pl.ds(i * 16, 16)] = jnp.zeros((16,), jnp.int32)
        return 0
    lax.fori_loop(0, NP // 16, zbody, 0)

    def cbody(i, _):
        sl = pl.ds(i * 16, 16)
        r = rows_v[sl]
        dup, last = plsc.scan_count(r)
        cbase = plsc.load_gather(cnt_v, [r])
        lrank_v[sl] = cbase + dup - 1
        plsc.store_scatter(cnt_v, [r], cbase + dup, mask=last)
        return 0
    lax.fori_loop(0, EB // 16, cbody, 0)

    pltpu.sync_copy(lrank_v, lrank_hbm.at[pl.ds(base, EB)])
    pltpu.sync_copy(cnt_v, blk_hbm.at[pl.ds(wid * NP, NP)])


@functools.partial(
    pl.kernel,
    out_type=(jax.ShapeDtypeStruct((NW * NP,), jnp.int32),
              jax.ShapeDtypeStruct((NP,), jnp.int32)),
    mesh=_mesh(),
    compiler_params=pltpu.CompilerParams(needs_layout_passes=False),
    scratch_types=[pltpu.VMEM((NW * NPT,), jnp.int32),
                   pltpu.VMEM((NW * NPT,), jnp.int32),
                   pltpu.VMEM((NPT,), jnp.int32)])
def _k2(blk_hbm, offs_hbm, ctot_hbm, blk_v, offs_v, ctot_v):
    wid = _wid()
    nb = wid * NPT
    for q in range(NW):
        pltpu.sync_copy(blk_hbm.at[pl.ds(q * NP + nb, NPT)],
                        blk_v.at[pl.ds(q * NPT, NPT)])

    def jbody(j, _):
        run = jnp.zeros((16,), jnp.int32)
        for q in range(NW):
            sl = pl.ds(q * NPT + j * 16, 16)
            offs_v[sl] = run
            run = run + blk_v[sl]
        ctot_v[pl.ds(j * 16, 16)] = run
        return 0
    lax.fori_loop(0, NPT // 16, jbody, 0)

    for q in range(NW):
        pltpu.sync_copy(offs_v.at[pl.ds(q * NPT, NPT)],
                        offs_hbm.at[pl.ds(q * NP + nb, NPT)])
    pltpu.sync_copy(ctot_v, ctot_hbm.at[pl.ds(nb, NPT)])


SLOT_TOTAL = NK + SLOT_PAD
STRIPE = SLOT_TOTAL // NS


@functools.partial(
    pl.kernel,
    out_type=jax.ShapeDtypeStruct((NC * SLOT_TOTAL,), jnp.int32),
    mesh=_mesh(),
    compiler_params=pltpu.CompilerParams(needs_layout_passes=False),
    scratch_types=[pltpu.VMEM((EBP,), jnp.int32),
                   pltpu.VMEM((EBP,), jnp.int32),
                   pltpu.VMEM((EBP,), jnp.int32),
                   pltpu.VMEM((NP,), jnp.int32),
                   pltpu.VMEM((80, 128), jnp.int32),
                   pltpu.VMEM((80, 128), jnp.int32),
                   pltpu.VMEM((STRIPE,), jnp.int32),
                   pltpu.VMEM_SHARED((SLOT_TOTAL,), jnp.int32)])
def _k3(rows_hbm, cols_hbm, lrank_hbm, offs_hbm, part_hbm,
        rows_v, cols_v, lrank_v, offs_v, idx_v, val_v, zbuf_v, shared):
    cid = lax.axis_index("c")
    sid = lax.axis_index("s")
    wid = sid * NC + cid
    base = wid * EB
    pltpu.sync_copy(rows_hbm.at[pl.ds(base, EB)], rows_v.at[pl.ds(0, EB)])
    pltpu.sync_copy(cols_hbm.at[pl.ds(base, EB)], cols_v.at[pl.ds(0, EB)])
    pltpu.sync_copy(lrank_hbm.at[pl.ds(base, EB)], lrank_v.at[pl.ds(0, EB)])
    pltpu.sync_copy(offs_hbm.at[pl.ds(wid * NP, NP)], offs_v)
    def zb(i, _):
        zbuf_v[pl.ds(i * 16, 16)] = jnp.zeros((16,), jnp.int32)
        return 0
    lax.fori_loop(0, STRIPE // 16, zb, 0)
    pltpu.sync_copy(zbuf_v, shared.at[pl.ds(sid * STRIPE, STRIPE)])
    for t in range((EBP - EB) // 16):
        rows_v[pl.ds(EB + t * 16, 16)] = jnp.zeros((16,), jnp.int32)

    def body(i, _):
        sl = pl.ds(i * 16, 16)
        r = rows_v[sl]
        lr = lrank_v[sl]
        c = cols_v[sl]
        off = plsc.load_gather(offs_v, [r])
        g = off + lr
        pos = i * 16 + lax.iota(jnp.int32, 16)
        sel = jnp.logical_and(g < K, pos < EB)
        slotpos = r * SLOTW + g
        dummy = NK + jnp.bitwise_and(pos, SLOT_PAD - 1)
        idx = jnp.where(sel, slotpos, dummy)
        row = i // 8
        col = (i % 8) * 16
        idx_v[row, pl.ds(col, 16)] = idx
        val_v[row, pl.ds(col, 16)] = c
        return 0
    lax.fori_loop(0, EBP // 16, body, 0)

    plsc.subcore_barrier()

    def sbody(j, _):
        pltpu.sync_copy(val_v.at[j], shared.at[idx_v.at[j]], add=True)
        return 0
    lax.fori_loop(0, 80, sbody, 0)

    plsc.subcore_barrier()
    pltpu.sync_copy(
        shared.at[pl.ds(sid * STRIPE, STRIPE)],
        part_hbm.at[pl.ds(cid * SLOT_TOTAL + sid * STRIPE, STRIPE)])


CHN = 16
NCH = NPT // CHN
NBUF = 2
GLP = (NCH + NBUF - 1) * CHN * K


DW = D // 2


@functools.partial(
    pl.kernel,
    out_type=jax.ShapeDtypeStruct((NP * D,), jnp.float32),
    mesh=_mesh(),
    compiler_params=pltpu.CompilerParams(needs_layout_passes=False),
    scratch_types=[pltpu.VMEM((NPT,), jnp.int32),
                   pltpu.VMEM((NPT,), jnp.float32),
                   pltpu.VMEM((NPT * SLOTW,), jnp.int32),
                   pltpu.VMEM((NPT * SLOTW,), jnp.int32),
                   pltpu.VMEM((GLP,), jnp.int32),
                   pltpu.VMEM((CHN * K, D), jnp.float32),
                   pltpu.VMEM((CHN * K, D), jnp.float32),
                   pltpu.VMEM((NPT * D,), jnp.float32),
                   pltpu.VMEM((D,), jnp.float32),
                   pltpu.SemaphoreType.DMA,
                   pltpu.SemaphoreType.DMA])
def _k4(ctot_hbm, part_hbm, xw_hbm, b_hbm, out_hbm,
        cnt_v, inv_v, slot_v, slot1_v, gl_v, rows0_v, rows1_v, out_v, b_v,
        sem0, sem1):
    wid = _wid()
    v0 = wid * NPT
    pltpu.sync_copy(ctot_hbm.at[pl.ds(v0, NPT)], cnt_v)
    pltpu.sync_copy(part_hbm.at[pl.ds(v0 * SLOTW, NPT * SLOTW)],
                    slot_v)
    pltpu.sync_copy(part_hbm.at[pl.ds(SLOT_TOTAL + v0 * SLOTW, NPT * SLOTW)],
                    slot1_v)
    pltpu.sync_copy(b_hbm, b_v)

    def mbody(i, _):
        sl = pl.ds(i * 16, 16)
        slot_v[sl] = slot_v[sl] + slot1_v[sl]
        return 0
    lax.fori_loop(0, NPT * SLOTW // 16, mbody, 0)

    def gbody(nc, _):
        sl = pl.ds(nc * 16, 16)
        ln = nc * 16 + lax.iota(jnp.int32, 16)
        cnt = cnt_v[sl]
        deg = jnp.minimum(cnt, K)
        degf = jnp.maximum(deg, 1).astype(jnp.float32)
        inv_v[sl] = 1.0 / degf
        nid = v0 + ln
        for j in range(K):
            sj = plsc.load_gather(slot_v, [ln * SLOTW + j])
            vj = jnp.where(deg > j, sj, DUMROW)
            if j == 0:
                vj = jnp.where(deg == 0, nid, vj)
            plsc.store_scatter(gl_v, [ln * K + j], vj)
        return 0
    lax.fori_loop(0, NPT // 16, gbody, 0)
    def pbody(i, _):
        base = NCH * CHN * K + i * 16
        gl_v[pl.ds(base, 16)] = N + jnp.bitwise_and(
            base + lax.iota(jnp.int32, 16), 127)
        return 0
    lax.fori_loop(0, (GLP - NCH * CHN * K) // 16, pbody, 0)

    def _gather(c, dst, sem):
        return pltpu.async_copy(
            xw_hbm.at[gl_v.at[pl.ds(c * (CHN * K), CHN * K)]], dst, sem)

    def _process(c, rows_v):
        for n in range(CHN):
            node = c * CHN + n
            inv16 = plsc.load_gather(inv_v, [jnp.full((16,), node, jnp.int32)])
            for k8 in range(D // 16):
                sl = pl.ds(k8 * 16, 16)
                acc = rows_v[n * K, sl]
                for r in range(1, K):
                    acc = acc + rows_v[n * K + r, sl]
                out_v[pl.ds(node * D + k8 * 16, 16)] = (
                    acc * inv16 + b_v[sl])

    bufs = (rows0_v, rows1_v)
    sems = (sem0, sem1)

    def _wait(c, dst, sem):
        pltpu.make_async_copy(
            xw_hbm.at[gl_v.at[pl.ds(c * (CHN * K), CHN * K)]], dst, sem).wait()

    for b in range(NBUF - 1):
        _gather(b, bufs[b], sems[b])

    def cbody(cc, _):
        c = cc * NBUF
        for b in range(NBUF):
            bn = (b + NBUF - 1) % NBUF
            _gather(c + b + NBUF - 1, bufs[bn], sems[bn])
            _wait(c + b, bufs[b], sems[b])
            _process(c + b, bufs[b])
        return 0
    lax.fori_loop(0, NCH // NBUF, cbody, 0)
    for b in range(NBUF - 1):
        _wait(NCH + b, bufs[b], sems[b])

    pltpu.sync_copy(out_v, out_hbm.at[pl.ds(v0 * D, NPT * D)])


def _mm_body(x_ref, w_ref, o_ref):
    o_ref[...] = lax.dot_general(
        x_ref[...], w_ref[...], (((1,), (1,)), ((), ())),
        preferred_element_type=jnp.float32)


def _matmul(x_pad, w):
    return pl.pallas_call(
        _mm_body,
        grid=(NP // 256,),
        in_specs=[pl.BlockSpec((256, D), lambda i: (i, 0)),
                  pl.BlockSpec((D, D), lambda i: (0, 0))],
        out_specs=pl.BlockSpec((256, D), lambda i: (i, 0)),
        out_shape=jax.ShapeDtypeStruct((NP, D), jnp.float32),
    )(x_pad, w)


def kernel(x, edge_index, W, b):
    rows = edge_index[0]
    cols = edge_index[1]
    x_pad = jnp.concatenate([x, jnp.zeros((NP - N, D), x.dtype)], axis=0)
    xw = _matmul(x_pad, W)
    xw_big = jnp.concatenate([xw, jnp.zeros((NP, D), jnp.float32)], axis=0)
    lrank, blk = _k1(rows)
    offs, ctot = _k2(blk)
    slot = _k3(rows, cols, lrank, offs)
    out = _k4(ctot, slot, xw_big, b)
    return out.reshape(NP, D)[:N]


# speedup vs baseline: 1.2127x; 1.2127x over previous
import functools

import jax
import jax.numpy as jnp
from jax import lax
from jax.experimental import pallas as pl
from jax.experimental.pallas import tpu as pltpu
from jax.experimental.pallas import tpu_sc as plsc

N = 10000
E = 320000
D = 128
K = 10

NC = 2
NS = 16
NW = NC * NS
EB = E // NW
EBP = EB + 240
NP = 10240
NPT = NP // NW
SLOTW = 16
NK = NP * SLOTW
SLOT_PAD = 8192
DUMROW = N


def _mesh():
    return plsc.VectorSubcoreMesh(
        core_axis_name="c", subcore_axis_name="s",
        num_cores=NC, num_subcores=NS)


def _wid():
    return lax.axis_index("s") * NC + lax.axis_index("c")


@functools.partial(
    pl.kernel,
    out_type=(jax.ShapeDtypeStruct((E,), jnp.int32),
              jax.ShapeDtypeStruct((NW * NP,), jnp.int32)),
    mesh=_mesh(),
    compiler_params=pltpu.CompilerParams(needs_layout_passes=False),
    scratch_types=[pltpu.VMEM((EB,), jnp.int32),
                   pltpu.VMEM((EB,), jnp.int32),
                   pltpu.VMEM((NP,), jnp.int32)])
def _k1(rows_hbm, lrank_hbm, blk_hbm, rows_v, lrank_v, cnt_v):
    wid = _wid()
    base = wid * EB
    pltpu.sync_copy(rows_hbm.at[pl.ds(base, EB)], rows_v)

    def zbody(i, _):
        cnt_v[pl.ds(i * 16, 16)] = jnp.zeros((16,), jnp.int32)
        return 0
    lax.fori_loop(0, NP // 16, zbody, 0)

    def cbody(i, _):
        sl = pl.ds(i * 16, 16)
        r = rows_v[sl]
        dup, last = plsc.scan_count(r)
        cbase = plsc.load_gather(cnt_v, [r])
        lrank_v[sl] = cbase + dup - 1
        plsc.store_scatter(cnt_v, [r], cbase + dup, mask=last)
        return 0
    lax.fori_loop(0, EB // 16, cbody, 0)

    pltpu.sync_copy(lrank_v, lrank_hbm.at[pl.ds(base, EB)])
    pltpu.sync_copy(cnt_v, blk_hbm.at[pl.ds(wid * NP, NP)])


@functools.partial(
    pl.kernel,
    out_type=(jax.ShapeDtypeStruct((NW * NP,), jnp.int32),
              jax.ShapeDtypeStruct((NP,), jnp.int32)),
    mesh=_mesh(),
    compiler_params=pltpu.CompilerParams(needs_layout_passes=False),
    scratch_types=[pltpu.VMEM((NW * NPT,), jnp.int32),
                   pltpu.VMEM((NW * NPT,), jnp.int32),
                   pltpu.VMEM((NPT,), jnp.int32),
                   pltpu.SemaphoreType.DMA])
def _k2(blk_hbm, offs_hbm, ctot_hbm, blk_v, offs_v, ctot_v, sem):
    wid = _wid()
    nb = wid * NPT
    for q in range(NW):
        pltpu.async_copy(blk_hbm.at[pl.ds(q * NP + nb, NPT)],
                         blk_v.at[pl.ds(q * NPT, NPT)], sem)
    for q in range(NW):
        pltpu.make_async_copy(blk_hbm.at[pl.ds(q * NP + nb, NPT)],
                              blk_v.at[pl.ds(q * NPT, NPT)], sem).wait()

    def jbody(j, _):
        run = jnp.zeros((16,), jnp.int32)
        for q in range(NW):
            sl = pl.ds(q * NPT + j * 16, 16)
            offs_v[sl] = run
            run = run + blk_v[sl]
        ctot_v[pl.ds(j * 16, 16)] = run
        return 0
    lax.fori_loop(0, NPT // 16, jbody, 0)

    for q in range(NW):
        pltpu.async_copy(offs_v.at[pl.ds(q * NPT, NPT)],
                         offs_hbm.at[pl.ds(q * NP + nb, NPT)], sem)
    pltpu.async_copy(ctot_v, ctot_hbm.at[pl.ds(nb, NPT)], sem)
    for q in range(NW):
        pltpu.make_async_copy(offs_v.at[pl.ds(q * NPT, NPT)],
                              offs_hbm.at[pl.ds(q * NP + nb, NPT)], sem).wait()
    pltpu.make_async_copy(ctot_v, ctot_hbm.at[pl.ds(nb, NPT)], sem).wait()


SLOT_TOTAL = NK + SLOT_PAD
STRIPE = SLOT_TOTAL // NS


@functools.partial(
    pl.kernel,
    out_type=jax.ShapeDtypeStruct((NC * SLOT_TOTAL,), jnp.int32),
    mesh=_mesh(),
    compiler_params=pltpu.CompilerParams(needs_layout_passes=False),
    scratch_types=[pltpu.VMEM((EBP,), jnp.int32),
                   pltpu.VMEM((EBP,), jnp.int32),
                   pltpu.VMEM((EBP,), jnp.int32),
                   pltpu.VMEM((NP,), jnp.int32),
                   pltpu.VMEM((80, 128), jnp.int32),
                   pltpu.VMEM((80, 128), jnp.int32),
                   pltpu.VMEM((STRIPE,), jnp.int32),
                   pltpu.VMEM_SHARED((SLOT_TOTAL,), jnp.int32),
                   pltpu.SemaphoreType.DMA])
def _k3(rows_hbm, cols_hbm, lrank_hbm, offs_hbm, part_hbm,
        rows_v, cols_v, lrank_v, offs_v, idx_v, val_v, zbuf_v, shared, sem):
    cid = lax.axis_index("c")
    sid = lax.axis_index("s")
    wid = sid * NC + cid
    base = wid * EB
    pltpu.async_copy(rows_hbm.at[pl.ds(base, EB)], rows_v.at[pl.ds(0, EB)], sem)
    pltpu.async_copy(cols_hbm.at[pl.ds(base, EB)], cols_v.at[pl.ds(0, EB)], sem)
    pltpu.async_copy(lrank_hbm.at[pl.ds(base, EB)], lrank_v.at[pl.ds(0, EB)],
                     sem)
    pltpu.async_copy(offs_hbm.at[pl.ds(wid * NP, NP)], offs_v, sem)
    pltpu.make_async_copy(rows_hbm.at[pl.ds(base, EB)],
                          rows_v.at[pl.ds(0, EB)], sem).wait()
    pltpu.make_async_copy(cols_hbm.at[pl.ds(base, EB)],
                          cols_v.at[pl.ds(0, EB)], sem).wait()
    pltpu.make_async_copy(lrank_hbm.at[pl.ds(base, EB)],
                          lrank_v.at[pl.ds(0, EB)], sem).wait()
    pltpu.make_async_copy(offs_hbm.at[pl.ds(wid * NP, NP)], offs_v,
                          sem).wait()
    def zb(i, _):
        zbuf_v[pl.ds(i * 16, 16)] = jnp.zeros((16,), jnp.int32)
        return 0
    lax.fori_loop(0, STRIPE // 16, zb, 0)
    pltpu.sync_copy(zbuf_v, shared.at[pl.ds(sid * STRIPE, STRIPE)])
    for t in range((EBP - EB) // 16):
        rows_v[pl.ds(EB + t * 16, 16)] = jnp.zeros((16,), jnp.int32)

    def body(i, _):
        sl = pl.ds(i * 16, 16)
        r = rows_v[sl]
        lr = lrank_v[sl]
        c = cols_v[sl]
        off = plsc.load_gather(offs_v, [r])
        g = off + lr
        pos = i * 16 + lax.iota(jnp.int32, 16)
        sel = jnp.logical_and(g < K, pos < EB)
        slotpos = r * SLOTW + g
        dummy = NK + jnp.bitwise_and(pos, SLOT_PAD - 1)
        idx = jnp.where(sel, slotpos, dummy)
        row = i // 8
        col = (i % 8) * 16
        idx_v[row, pl.ds(col, 16)] = idx
        val_v[row, pl.ds(col, 16)] = c
        return 0
    lax.fori_loop(0, EBP // 16, body, 0)

    plsc.subcore_barrier()

    def sbody(j, _):
        pltpu.async_copy(val_v.at[j], shared.at[idx_v.at[j]], sem, add=True)
        return 0
    lax.fori_loop(0, 80, sbody, 0)

    def dbody(j, _):
        pltpu.make_async_copy(val_v.at[j], shared.at[idx_v.at[j]], sem).wait()
        return 0
    lax.fori_loop(0, 80, dbody, 0)

    plsc.subcore_barrier()
    pltpu.sync_copy(
        shared.at[pl.ds(sid * STRIPE, STRIPE)],
        part_hbm.at[pl.ds(cid * SLOT_TOTAL + sid * STRIPE, STRIPE)])


CHN = 8
NCH = NPT // CHN
NBUF = 2
GLP = (NCH + NBUF - 1) * CHN * K


DW = D // 2


@functools.partial(
    pl.kernel,
    out_type=jax.ShapeDtypeStruct((NP * D,), jnp.float32),
    mesh=_mesh(),
    compiler_params=pltpu.CompilerParams(needs_layout_passes=False),
    scratch_types=[pltpu.VMEM((NPT,), jnp.int32),
                   pltpu.VMEM((NPT,), jnp.float32),
                   pltpu.VMEM((NPT * SLOTW,), jnp.int32),
                   pltpu.VMEM((NPT * SLOTW,), jnp.int32),
                   pltpu.VMEM((GLP,), jnp.int32),
                   pltpu.VMEM((CHN * K, D), jnp.float32),
                   pltpu.VMEM((CHN * K, D), jnp.float32),
                   pltpu.VMEM((NPT * D,), jnp.float32),
                   pltpu.VMEM((D,), jnp.float32),
                   pltpu.SemaphoreType.DMA,
                   pltpu.SemaphoreType.DMA])
def _k4(ctot_hbm, part_hbm, xw_hbm, b_hbm, out_hbm,
        cnt_v, inv_v, slot_v, slot1_v, gl_v, rows0_v, rows1_v, out_v, b_v,
        sem0, sem1):
    wid = _wid()
    v0 = wid * NPT
    pltpu.sync_copy(ctot_hbm.at[pl.ds(v0, NPT)], cnt_v)
    pltpu.sync_copy(part_hbm.at[pl.ds(v0 * SLOTW, NPT * SLOTW)],
                    slot_v)
    pltpu.sync_copy(part_hbm.at[pl.ds(SLOT_TOTAL + v0 * SLOTW, NPT * SLOTW)],
                    slot1_v)
    pltpu.sync_copy(b_hbm, b_v)

    def mbody(i, _):
        sl = pl.ds(i * 16, 16)
        slot_v[sl] = slot_v[sl] + slot1_v[sl]
        return 0
    lax.fori_loop(0, NPT * SLOTW // 16, mbody, 0)

    def gbody(nc, _):
        sl = pl.ds(nc * 16, 16)
        ln = nc * 16 + lax.iota(jnp.int32, 16)
        cnt = cnt_v[sl]
        deg = jnp.minimum(cnt, K)
        degf = jnp.maximum(deg, 1).astype(jnp.float32)
        inv_v[sl] = 1.0 / degf
        nid = v0 + ln
        for j in range(K):
            sj = plsc.load_gather(slot_v, [ln * SLOTW + j])
            vj = jnp.where(deg > j, sj, DUMROW)
            if j == 0:
                vj = jnp.where(deg == 0, nid, vj)
            plsc.store_scatter(gl_v, [ln * K + j], vj)
        return 0
    lax.fori_loop(0, NPT // 16, gbody, 0)
    def pbody(i, _):
        base = NCH * CHN * K + i * 16
        gl_v[pl.ds(base, 16)] = N + jnp.bitwise_and(
            base + lax.iota(jnp.int32, 16), 127)
        return 0
    lax.fori_loop(0, (GLP - NCH * CHN * K) // 16, pbody, 0)

    def _gather(c, dst, sem):
        return pltpu.async_copy(
            xw_hbm.at[gl_v.at[pl.ds(c * (CHN * K), CHN * K)]], dst, sem)

    def _process(c, rows_v):
        for n in range(CHN):
            node = c * CHN + n
            inv16 = plsc.load_gather(inv_v, [jnp.full((16,), node, jnp.int32)])
            for k8 in range(D // 16):
                sl = pl.ds(k8 * 16, 16)
                acc = rows_v[n * K, sl]
                for r in range(1, K):
                    acc = acc + rows_v[n * K + r, sl]
                out_v[pl.ds(node * D + k8 * 16, 16)] = (
                    acc * inv16 + b_v[sl])

    bufs = (rows0_v, rows1_v)
    sems = (sem0, sem1)

    def _wait(c, dst, sem):
        pltpu.make_async_copy(
            xw_hbm.at[gl_v.at[pl.ds(c * (CHN * K), CHN * K)]], dst, sem).wait()

    for b in range(NBUF - 1):
        _gather(b, bufs[b], sems[b])

    def cbody(cc, _):
        c = cc * NBUF
        for b in range(NBUF):
            bn = (b + NBUF - 1) % NBUF
            _gather(c + b + NBUF - 1, bufs[bn], sems[bn])
            _wait(c + b, bufs[b], sems[b])
            _process(c + b, bufs[b])
        return 0
    lax.fori_loop(0, NCH // NBUF, cbody, 0)
    for b in range(NBUF - 1):
        _wait(NCH + b, bufs[b], sems[b])

    pltpu.sync_copy(out_v, out_hbm.at[pl.ds(v0 * D, NPT * D)])


def _mm_body(x_ref, w_ref, o_ref):
    o_ref[...] = lax.dot_general(
        x_ref[...], w_ref[...], (((1,), (1,)), ((), ())),
        preferred_element_type=jnp.float32)


def _matmul(x_pad, w):
    return pl.pallas_call(
        _mm_body,
        grid=(NP // 256,),
        in_specs=[pl.BlockSpec((256, D), lambda i: (i, 0)),
                  pl.BlockSpec((D, D), lambda i: (0, 0))],
        out_specs=pl.BlockSpec((256, D), lambda i: (i, 0)),
        out_shape=jax.ShapeDtypeStruct((NP, D), jnp.float32),
    )(x_pad, w)


def kernel(x, edge_index, W, b):
    rows = edge_index[0]
    cols = edge_index[1]
    x_pad = jnp.concatenate([x, jnp.zeros((NP - N, D), x.dtype)], axis=0)
    xw = _matmul(x_pad, W)
    lrank, blk = _k1(rows)
    offs, ctot = _k2(blk)
    slot = _k3(rows, cols, lrank, offs)
    out = _k4(ctot, slot, xw, b)
    return out.reshape(NP, D)[:N]


# speedup vs baseline: 1.2444x; 1.0262x over previous
import functools

import jax
import jax.numpy as jnp
from jax import lax
from jax.experimental import pallas as pl
from jax.experimental.pallas import tpu as pltpu
from jax.experimental.pallas import tpu_sc as plsc

N = 10000
E = 320000
D = 128
K = 10

NC = 2
NS = 16
NW = NC * NS
EB = E // NW
EBP = EB + 240
NP = 10240
NPT = NP // NW
SLOTW = 16
NK = NP * SLOTW
SLOT_PAD = 8192
DUMROW = N


def _mesh():
    return plsc.VectorSubcoreMesh(
        core_axis_name="c", subcore_axis_name="s",
        num_cores=NC, num_subcores=NS)


def _wid():
    return lax.axis_index("s") * NC + lax.axis_index("c")


@functools.partial(
    pl.kernel,
    out_type=(jax.ShapeDtypeStruct((E,), jnp.int32),
              jax.ShapeDtypeStruct((NW * NP,), jnp.int32)),
    mesh=_mesh(),
    compiler_params=pltpu.CompilerParams(needs_layout_passes=False),
    scratch_types=[pltpu.VMEM((EB,), jnp.int32),
                   pltpu.VMEM((EB,), jnp.int32),
                   pltpu.VMEM((NP,), jnp.int32)])
def _k1(rows_hbm, lrank_hbm, blk_hbm, rows_v, lrank_v, cnt_v):
    wid = _wid()
    base = wid * EB
    pltpu.sync_copy(rows_hbm.at[pl.ds(base, EB)], rows_v)

    def zbody(i, _):
        cnt_v[pl.ds(i * 16, 16)] = jnp.zeros((16,), jnp.int32)
        return 0
    lax.fori_loop(0, NP // 16, zbody, 0)

    def cbody(i, _):
        sl = pl.ds(i * 16, 16)
        r = rows_v[sl]
        dup, last = plsc.scan_count(r)
        cbase = plsc.load_gather(cnt_v, [r])
        lrank_v[sl] = cbase + dup - 1
        plsc.store_scatter(cnt_v, [r], cbase + dup, mask=last)
        return 0
    lax.fori_loop(0, EB // 16, cbody, 0)

    pltpu.sync_copy(lrank_v, lrank_hbm.at[pl.ds(base, EB)])
    pltpu.sync_copy(cnt_v, blk_hbm.at[pl.ds(wid * NP, NP)])


@functools.partial(
    pl.kernel,
    out_type=(jax.ShapeDtypeStruct((NW * NP,), jnp.int32),
              jax.ShapeDtypeStruct((NP,), jnp.int32)),
    mesh=_mesh(),
    compiler_params=pltpu.CompilerParams(needs_layout_passes=False),
    scratch_types=[pltpu.VMEM((NW * NPT,), jnp.int32),
                   pltpu.VMEM((NW * NPT,), jnp.int32),
                   pltpu.VMEM((NPT,), jnp.int32),
                   pltpu.SemaphoreType.DMA])
def _k2(blk_hbm, offs_hbm, ctot_hbm, blk_v, offs_v, ctot_v, sem):
    wid = _wid()
    nb = wid * NPT
    for q in range(NW):
        pltpu.async_copy(blk_hbm.at[pl.ds(q * NP + nb, NPT)],
                         blk_v.at[pl.ds(q * NPT, NPT)], sem)
    for q in range(NW):
        pltpu.make_async_copy(blk_hbm.at[pl.ds(q * NP + nb, NPT)],
                              blk_v.at[pl.ds(q * NPT, NPT)], sem).wait()

    def jbody(j, _):
        run = jnp.zeros((16,), jnp.int32)
        for q in range(NW):
            sl = pl.ds(q * NPT + j * 16, 16)
            offs_v[sl] = run
            run = run + blk_v[sl]
        ctot_v[pl.ds(j * 16, 16)] = run
        return 0
    lax.fori_loop(0, NPT // 16, jbody, 0)

    for q in range(NW):
        pltpu.async_copy(offs_v.at[pl.ds(q * NPT, NPT)],
                         offs_hbm.at[pl.ds(q * NP + nb, NPT)], sem)
    pltpu.async_copy(ctot_v, ctot_hbm.at[pl.ds(nb, NPT)], sem)
    for q in range(NW):
        pltpu.make_async_copy(offs_v.at[pl.ds(q * NPT, NPT)],
                              offs_hbm.at[pl.ds(q * NP + nb, NPT)], sem).wait()
    pltpu.make_async_copy(ctot_v, ctot_hbm.at[pl.ds(nb, NPT)], sem).wait()


SLOT_TOTAL = NK + SLOT_PAD
STRIPE = SLOT_TOTAL // NS


@functools.partial(
    pl.kernel,
    out_type=jax.ShapeDtypeStruct((NC * SLOT_TOTAL,), jnp.int32),
    mesh=_mesh(),
    compiler_params=pltpu.CompilerParams(needs_layout_passes=False),
    scratch_types=[pltpu.VMEM((EBP,), jnp.int32),
                   pltpu.VMEM((EBP,), jnp.int32),
                   pltpu.VMEM((EBP,), jnp.int32),
                   pltpu.VMEM((NP,), jnp.int32),
                   pltpu.VMEM((80, 128), jnp.int32),
                   pltpu.VMEM((80, 128), jnp.int32),
                   pltpu.VMEM((STRIPE,), jnp.int32),
                   pltpu.VMEM_SHARED((SLOT_TOTAL,), jnp.int32),
                   pltpu.SemaphoreType.DMA])
def _k3(rows_hbm, cols_hbm, lrank_hbm, offs_hbm, part_hbm,
        rows_v, cols_v, lrank_v, offs_v, idx_v, val_v, zbuf_v, shared, sem):
    cid = lax.axis_index("c")
    sid = lax.axis_index("s")
    wid = sid * NC + cid
    base = wid * EB
    pltpu.async_copy(rows_hbm.at[pl.ds(base, EB)], rows_v.at[pl.ds(0, EB)], sem)
    pltpu.async_copy(cols_hbm.at[pl.ds(base, EB)], cols_v.at[pl.ds(0, EB)], sem)
    pltpu.async_copy(lrank_hbm.at[pl.ds(base, EB)], lrank_v.at[pl.ds(0, EB)],
                     sem)
    pltpu.async_copy(offs_hbm.at[pl.ds(wid * NP, NP)], offs_v, sem)
    pltpu.make_async_copy(rows_hbm.at[pl.ds(base, EB)],
                          rows_v.at[pl.ds(0, EB)], sem).wait()
    pltpu.make_async_copy(cols_hbm.at[pl.ds(base, EB)],
                          cols_v.at[pl.ds(0, EB)], sem).wait()
    pltpu.make_async_copy(lrank_hbm.at[pl.ds(base, EB)],
                          lrank_v.at[pl.ds(0, EB)], sem).wait()
    pltpu.make_async_copy(offs_hbm.at[pl.ds(wid * NP, NP)], offs_v,
                          sem).wait()
    def zb(i, _):
        zbuf_v[pl.ds(i * 16, 16)] = jnp.zeros((16,), jnp.int32)
        return 0
    lax.fori_loop(0, STRIPE // 16, zb, 0)
    pltpu.sync_copy(zbuf_v, shared.at[pl.ds(sid * STRIPE, STRIPE)])
    for t in range((EBP - EB) // 16):
        rows_v[pl.ds(EB + t * 16, 16)] = jnp.zeros((16,), jnp.int32)

    def body(i, _):
        sl = pl.ds(i * 16, 16)
        r = rows_v[sl]
        lr = lrank_v[sl]
        c = cols_v[sl]
        off = plsc.load_gather(offs_v, [r])
        g = off + lr
        pos = i * 16 + lax.iota(jnp.int32, 16)
        sel = jnp.logical_and(g < K, pos < EB)
        slotpos = r * SLOTW + g
        dummy = NK + jnp.bitwise_and(pos, SLOT_PAD - 1)
        idx = jnp.where(sel, slotpos, dummy)
        row = i // 8
        col = (i % 8) * 16
        idx_v[row, pl.ds(col, 16)] = idx
        val_v[row, pl.ds(col, 16)] = c
        return 0
    lax.fori_loop(0, EBP // 16, body, 0)

    plsc.subcore_barrier()

    def sbody(j, _):
        pltpu.async_copy(val_v.at[j], shared.at[idx_v.at[j]], sem, add=True)
        return 0
    lax.fori_loop(0, 80, sbody, 0)

    def dbody(j, _):
        pltpu.make_async_copy(val_v.at[j], shared.at[idx_v.at[j]], sem).wait()
        return 0
    lax.fori_loop(0, 80, dbody, 0)

    plsc.subcore_barrier()
    pltpu.sync_copy(
        shared.at[pl.ds(sid * STRIPE, STRIPE)],
        part_hbm.at[pl.ds(cid * SLOT_TOTAL + sid * STRIPE, STRIPE)])


CHN = 8
NCH = NPT // CHN
NBUF = 2
GLP = (NCH + NBUF - 1) * CHN * K


DW = D // 2


@functools.partial(
    pl.kernel,
    out_type=jax.ShapeDtypeStruct((NP * D,), jnp.float32),
    mesh=_mesh(),
    compiler_params=pltpu.CompilerParams(needs_layout_passes=False),
    scratch_types=[pltpu.VMEM((NPT,), jnp.int32),
                   pltpu.VMEM((NPT,), jnp.float32),
                   pltpu.VMEM((NPT * SLOTW,), jnp.int32),
                   pltpu.VMEM((NPT * SLOTW,), jnp.int32),
                   pltpu.VMEM((GLP,), jnp.int32),
                   pltpu.VMEM((CHN * K, D), jnp.float32),
                   pltpu.VMEM((CHN * K, D), jnp.float32),
                   pltpu.VMEM((NPT * D,), jnp.float32),
                   pltpu.VMEM((D,), jnp.float32),
                   pltpu.SemaphoreType.DMA,
                   pltpu.SemaphoreType.DMA])
def _k4(ctot_hbm, part_hbm, xw_hbm, xw2_hbm, b_hbm, out_hbm,
        cnt_v, inv_v, slot_v, slot1_v, gl_v, rows0_v, rows1_v, out_v, b_v,
        sem0, sem1):
    wid = _wid()
    v0 = wid * NPT
    pltpu.sync_copy(ctot_hbm.at[pl.ds(v0, NPT)], cnt_v)
    pltpu.sync_copy(part_hbm.at[pl.ds(v0 * SLOTW, NPT * SLOTW)],
                    slot_v)
    pltpu.sync_copy(part_hbm.at[pl.ds(SLOT_TOTAL + v0 * SLOTW, NPT * SLOTW)],
                    slot1_v)
    pltpu.sync_copy(b_hbm, b_v)

    def mbody(i, _):
        sl = pl.ds(i * 16, 16)
        slot_v[sl] = slot_v[sl] + slot1_v[sl]
        return 0
    lax.fori_loop(0, NPT * SLOTW // 16, mbody, 0)

    def gbody(nc, _):
        sl = pl.ds(nc * 16, 16)
        ln = nc * 16 + lax.iota(jnp.int32, 16)
        cnt = cnt_v[sl]
        deg = jnp.minimum(cnt, K)
        degf = jnp.maximum(deg, 1).astype(jnp.float32)
        inv_v[sl] = 1.0 / degf
        nid = v0 + ln
        for j in range(K):
            sj = plsc.load_gather(slot_v, [ln * SLOTW + j])
            vj = jnp.where(deg > j, sj, DUMROW)
            if j == 0:
                vj = jnp.where(deg == 0, nid, vj)
            plsc.store_scatter(gl_v, [ln * K + j], vj)
        return 0
    lax.fori_loop(0, NPT // 16, gbody, 0)
    def pbody(i, _):
        base = NCH * CHN * K + i * 16
        gl_v[pl.ds(base, 16)] = N + jnp.bitwise_and(
            base + lax.iota(jnp.int32, 16), 127)
        return 0
    lax.fori_loop(0, (GLP - NCH * CHN * K) // 16, pbody, 0)

    def _gather(c, dst, sem, src_hbm):
        return pltpu.async_copy(
            src_hbm.at[gl_v.at[pl.ds(c * (CHN * K), CHN * K)]], dst, sem)

    def _process(c, rows_v):
        for n in range(CHN):
            node = c * CHN + n
            inv16 = plsc.load_gather(inv_v, [jnp.full((16,), node, jnp.int32)])
            for k8 in range(D // 16):
                sl = pl.ds(k8 * 16, 16)
                acc = rows_v[n * K, sl]
                for r in range(1, K):
                    acc = acc + rows_v[n * K + r, sl]
                out_v[pl.ds(node * D + k8 * 16, 16)] = (
                    acc * inv16 + b_v[sl])

    bufs = (rows0_v, rows1_v)
    sems = (sem0, sem1)

    def _wait(c, dst, sem, src_hbm):
        pltpu.make_async_copy(
            src_hbm.at[gl_v.at[pl.ds(c * (CHN * K), CHN * K)]], dst,
            sem).wait()

    srcs = (xw_hbm, xw2_hbm)

    for b in range(NBUF - 1):
        _gather(b, bufs[b], sems[b], srcs[b % 2])

    def cbody(cc, _):
        c = cc * NBUF
        for b in range(NBUF):
            bn = (b + NBUF - 1) % NBUF
            cg = c + b + NBUF - 1
            _gather(cg, bufs[bn], sems[bn], srcs[(b + NBUF - 1) % 2])
            _wait(c + b, bufs[b], sems[b], srcs[b % 2])
            _process(c + b, bufs[b])
        return 0
    lax.fori_loop(0, NCH // NBUF, cbody, 0)
    for b in range(NBUF - 1):
        _wait(NCH + b, bufs[b], sems[b], srcs[b % 2])

    pltpu.sync_copy(out_v, out_hbm.at[pl.ds(v0 * D, NPT * D)])


def _mm_body(x_ref, w_ref, o_ref):
    o_ref[...] = lax.dot_general(
        x_ref[...], w_ref[...], (((1,), (1,)), ((), ())),
        preferred_element_type=jnp.float32)


def _matmul(x_pad, w):
    return pl.pallas_call(
        _mm_body,
        grid=(NP // 256,),
        in_specs=[pl.BlockSpec((256, D), lambda i: (i, 0)),
                  pl.BlockSpec((D, D), lambda i: (0, 0))],
        out_specs=pl.BlockSpec((256, D), lambda i: (i, 0)),
        out_shape=jax.ShapeDtypeStruct((NP, D), jnp.float32),
    )(x_pad, w)


def kernel(x, edge_index, W, b):
    rows = edge_index[0]
    cols = edge_index[1]
    x_pad = jnp.concatenate([x, jnp.zeros((NP - N, D), x.dtype)], axis=0)
    xw = _matmul(x_pad, W)
    xw2 = jnp.concatenate([xw, jnp.zeros((NP, D), jnp.float32)], axis=0)
    lrank, blk = _k1(rows)
    offs, ctot = _k2(blk)
    slot = _k3(rows, cols, lrank, offs)
    out = _k4(ctot, slot, xw, xw2, b)
    return out.reshape(NP, D)[:N]
